# baseline (speedup 1.0000x reference)
import functools

import jax
import jax.numpy as jnp
from jax import lax
from jax.experimental import pallas as pl
from jax.experimental.pallas import tpu as pltpu
from jax.experimental.pallas import tpu_sc as plsc

B = 16384
D = 32
N_BUCKETS = 2000
NB_PAD = 2048
NW = 32
ROWS = B // NW
CHUNK = 128
NCHUNK = ROWS // CHUNK
L = 16


def _body(uid_hbm, ts_hbm, age_hbm, utab_hbm, ttab_hbm, bounds_hbm,
          stats_hbm, out_hbm,
          uid_v, ts_v, age_v, bidx_v, bounds_v, stats_v,
          urows_v, trows_v, agecol_v, sem, gsem, osem):
    wid = lax.axis_index("s") * 2 + lax.axis_index("c")
    base = wid * ROWS

    stage = [
        pltpu.async_copy(uid_hbm.at[pl.ds(base, ROWS)], uid_v, sem),
        pltpu.async_copy(ts_hbm.at[pl.ds(base, ROWS)], ts_v, sem),
        pltpu.async_copy(age_hbm.at[pl.ds(base, ROWS)], age_v, sem),
        pltpu.async_copy(bounds_hbm, bounds_v, sem),
        pltpu.async_copy(stats_hbm, stats_v, sem),
    ]
    for c in stage:
        c.wait()

    ucopies = []
    for j in range(NCHUNK):
        ucopies.append(pltpu.async_copy(
            utab_hbm.at[uid_v.at[pl.ds(j * CHUNK, CHUNK)]],
            urows_v.at[pl.ds(j * CHUNK, CHUNK)], gsem))

    mean = stats_v[pl.ds(0, L)]
    std = stats_v[pl.ds(L, L)]
    lanes = lax.iota(jnp.int32, L)
    zcol = jnp.zeros((L,), jnp.int32)

    def bucketize(i, _):
        t = ts_v[pl.ds(i * L, L)].astype(jnp.float32)
        pos = jnp.zeros((L,), jnp.int32)
        step = NB_PAD // 2
        while step >= 1:
            probe = plsc.load_gather(bounds_v, [pos + (step - 1)])
            pos = jnp.where(probe <= t, pos + step, pos)
            step //= 2
        bidx_v[pl.ds(i * L, L)] = pos
        a = age_v[pl.ds(i * L, L)]
        plsc.store_scatter(agecol_v, [i * L + lanes, zcol], (a - mean) / std)
        return 0

    tcopies = []
    for blk in range(NCHUNK):
        lax.fori_loop(blk * (CHUNK // L), (blk + 1) * (CHUNK // L),
                      bucketize, 0, unroll=4)
        tcopies.append(pltpu.async_copy(
            ttab_hbm.at[bidx_v.at[pl.ds(blk * CHUNK, CHUNK)]],
            trows_v.at[pl.ds(blk * CHUNK, CHUNK)], gsem))

    ow = [pltpu.async_copy(agecol_v,
                           out_hbm.at[pl.ds(base, ROWS), pl.ds(2 * D, 1)],
                           osem)]
    for c in ucopies:
        c.wait()
    ow.append(pltpu.async_copy(
        urows_v, out_hbm.at[pl.ds(base, ROWS), pl.ds(0, D)], osem))
    for c in tcopies:
        c.wait()
    ow.append(pltpu.async_copy(
        trows_v, out_hbm.at[pl.ds(base, ROWS), pl.ds(D, D)], osem))
    for c in ow:
        c.wait()


@jax.jit
def _run(user_id, timestamp, bucketized_user_age, user_table, ts_table,
         bounds_pad, stats):
    mesh = plsc.VectorSubcoreMesh(core_axis_name="c", subcore_axis_name="s")
    kern = pl.kernel(
        _body,
        out_type=jax.ShapeDtypeStruct((B, 2 * D + 1), jnp.float32),
        mesh=mesh,
        scratch_types=[
            pltpu.VMEM((ROWS,), jnp.int32),
            pltpu.VMEM((ROWS,), jnp.int32),
            pltpu.VMEM((ROWS,), jnp.float32),
            pltpu.VMEM((ROWS,), jnp.int32),
            pltpu.VMEM((NB_PAD,), jnp.float32),
            pltpu.VMEM((2 * L,), jnp.float32),
            pltpu.VMEM((ROWS, D), jnp.float32),
            pltpu.VMEM((ROWS, D), jnp.float32),
            pltpu.VMEM((ROWS, 1), jnp.float32),
            pltpu.SemaphoreType.DMA,
            pltpu.SemaphoreType.DMA,
            pltpu.SemaphoreType.DMA,
        ],
        compiler_params=pltpu.CompilerParams(use_tc_tiling_on_sc=False,
                                             needs_layout_passes=False),
    )
    return kern(user_id, timestamp, bucketized_user_age, user_table,
                ts_table, bounds_pad, stats)


def kernel(user_id, timestamp, bucketized_user_age, user_table, ts_table,
           ts_boundaries, age_mean, age_std):
    uid = user_id.astype(jnp.int32)
    ts = timestamp.astype(jnp.int32)
    bounds_pad = jnp.concatenate(
        [ts_boundaries.astype(jnp.float32),
         jnp.full((NB_PAD - N_BUCKETS,), 3.0e38, jnp.float32)])
    stats = jnp.concatenate([
        jnp.full((L,), age_mean, jnp.float32),
        jnp.full((L,), age_std, jnp.float32)])
    return _run(uid, ts, bucketized_user_age.astype(jnp.float32),
                user_table.astype(jnp.float32), ts_table.astype(jnp.float32),
                bounds_pad, stats)


# speedup vs baseline: 1.0074x; 1.0074x over previous
import functools

import jax
import jax.numpy as jnp
from jax import lax
from jax.experimental import pallas as pl
from jax.experimental.pallas import tpu as pltpu
from jax.experimental.pallas import tpu_sc as plsc

B = 16384
D = 32
N_BUCKETS = 2000
NB_PAD = 2048
NW = 32
ROWS = B // NW
CHUNK = 128
NCHUNK = ROWS // CHUNK
L = 16


def _body(uid_hbm, ts_hbm, age_hbm, utab_hbm, ttab_hbm, bounds_hbm,
          stats_hbm, out_hbm,
          uid_v, ts_v, age_v, bidx_v, bounds_v, stats_v,
          urows_v, trows_v, agecol_v, sem, gsem, osem):
    wid = lax.axis_index("s") * 2 + lax.axis_index("c")
    base = wid * ROWS

    stage = [
        pltpu.async_copy(uid_hbm.at[pl.ds(base, ROWS)], uid_v, sem),
        pltpu.async_copy(ts_hbm.at[pl.ds(base, ROWS)], ts_v, sem),
        pltpu.async_copy(age_hbm.at[pl.ds(base, ROWS)], age_v, sem),
        pltpu.async_copy(bounds_hbm, bounds_v, sem),
        pltpu.async_copy(stats_hbm, stats_v, sem),
    ]
    for c in stage:
        c.wait()

    ucopies = []
    for j in range(NCHUNK):
        ucopies.append(pltpu.async_copy(
            utab_hbm.at[uid_v.at[pl.ds(j * CHUNK, CHUNK)]],
            urows_v.at[pl.ds(j * CHUNK, CHUNK)], gsem))

    mean = stats_v[pl.ds(0, L)]
    std = stats_v[pl.ds(L, L)]
    lanes = lax.iota(jnp.int32, L)
    zcol = jnp.zeros((L,), jnp.int32)

    def bucketize(i, _):
        t = ts_v[pl.ds(i * L, L)].astype(jnp.float32)
        pos = jnp.zeros((L,), jnp.int32)
        step = NB_PAD // 2
        while step >= 1:
            probe = plsc.load_gather(bounds_v, [pos + (step - 1)])
            pos = jnp.where(probe <= t, pos + step, pos)
            step //= 2
        bidx_v[pl.ds(i * L, L)] = pos
        a = age_v[pl.ds(i * L, L)]
        plsc.store_scatter(agecol_v, [i * L + lanes, zcol], (a - mean) / std)
        return 0

    tcopies = []
    for blk in range(NCHUNK):
        lax.fori_loop(blk * (CHUNK // L), (blk + 1) * (CHUNK // L),
                      bucketize, 0, unroll=2)
        tcopies.append(pltpu.async_copy(
            ttab_hbm.at[bidx_v.at[pl.ds(blk * CHUNK, CHUNK)]],
            trows_v.at[pl.ds(blk * CHUNK, CHUNK)], gsem))

    ow = [pltpu.async_copy(agecol_v,
                           out_hbm.at[pl.ds(base, ROWS), pl.ds(2 * D, 1)],
                           osem)]
    for c in ucopies:
        c.wait()
    ow.append(pltpu.async_copy(
        urows_v, out_hbm.at[pl.ds(base, ROWS), pl.ds(0, D)], osem))
    for c in tcopies:
        c.wait()
    ow.append(pltpu.async_copy(
        trows_v, out_hbm.at[pl.ds(base, ROWS), pl.ds(D, D)], osem))
    for c in ow:
        c.wait()


@jax.jit
def _run(user_id, timestamp, bucketized_user_age, user_table, ts_table,
         bounds_pad, stats):
    mesh = plsc.VectorSubcoreMesh(core_axis_name="c", subcore_axis_name="s")
    kern = pl.kernel(
        _body,
        out_type=jax.ShapeDtypeStruct((B, 2 * D + 1), jnp.float32),
        mesh=mesh,
        scratch_types=[
            pltpu.VMEM((ROWS,), jnp.int32),
            pltpu.VMEM((ROWS,), jnp.int32),
            pltpu.VMEM((ROWS,), jnp.float32),
            pltpu.VMEM((ROWS,), jnp.int32),
            pltpu.VMEM((NB_PAD,), jnp.float32),
            pltpu.VMEM((2 * L,), jnp.float32),
            pltpu.VMEM((ROWS, D), jnp.float32),
            pltpu.VMEM((ROWS, D), jnp.float32),
            pltpu.VMEM((ROWS, 1), jnp.float32),
            pltpu.SemaphoreType.DMA,
            pltpu.SemaphoreType.DMA,
            pltpu.SemaphoreType.DMA,
        ],
        compiler_params=pltpu.CompilerParams(use_tc_tiling_on_sc=False,
                                             needs_layout_passes=False),
    )
    return kern(user_id, timestamp, bucketized_user_age, user_table,
                ts_table, bounds_pad, stats)


def kernel(user_id, timestamp, bucketized_user_age, user_table, ts_table,
           ts_boundaries, age_mean, age_std):
    uid = user_id.astype(jnp.int32)
    ts = timestamp.astype(jnp.int32)
    bounds_pad = jnp.concatenate(
        [ts_boundaries.astype(jnp.float32),
         jnp.full((NB_PAD - N_BUCKETS,), 3.0e38, jnp.float32)])
    stats = jnp.concatenate([
        jnp.full((L,), age_mean, jnp.float32),
        jnp.full((L,), age_std, jnp.float32)])
    return _run(uid, ts, bucketized_user_age.astype(jnp.float32),
                user_table.astype(jnp.float32), ts_table.astype(jnp.float32),
                bounds_pad, stats)
